# Initial kernel scaffold; baseline (speedup 1.0000x reference)
import jax
import jax.numpy as jnp
from jax.experimental import pallas as pl


def kernel(word_emb, pos_emb, type_emb, emb_gamma, emb_beta, w_pool, b_pool, input_ids, token_type_ids, attention_mask, L0_w_qkv, L0_b_qkv, L0_w_ao, L0_b_ao, L0_g_ao, L0_be_ao, L0_w_i, L0_b_i, L0_w_o, L0_b_o, L0_g_o, L0_be_o, L1_w_qkv, L1_b_qkv, L1_w_ao, L1_b_ao, L1_g_ao, L1_be_ao, L1_w_i, L1_b_i, L1_w_o, L1_b_o, L1_g_o, L1_be_o, L2_w_qkv, L2_b_qkv, L2_w_ao, L2_b_ao, L2_g_ao, L2_be_ao, L2_w_i, L2_b_i, L2_w_o, L2_b_o, L2_g_o, L2_be_o, L3_w_qkv, L3_b_qkv, L3_w_ao, L3_b_ao, L3_g_ao, L3_be_ao, L3_w_i, L3_b_i, L3_w_o, L3_b_o, L3_g_o, L3_be_o, L4_w_qkv, L4_b_qkv, L4_w_ao, L4_b_ao, L4_g_ao, L4_be_ao, L4_w_i, L4_b_i, L4_w_o, L4_b_o, L4_g_o, L4_be_o, L5_w_qkv, L5_b_qkv, L5_w_ao, L5_b_ao, L5_g_ao, L5_be_ao, L5_w_i, L5_b_i, L5_w_o, L5_b_o, L5_g_o, L5_be_o):
    raise NotImplementedError("write your pallas kernel here")


# speedup vs baseline: 2.2560x; 2.2560x over previous
import functools
import math

import jax
import jax.numpy as jnp
from jax.experimental import pallas as pl
from jax.experimental.pallas import tpu as pltpu

_EPS = 1e-12
_VMEM_LIMIT = 64 * 1024 * 1024
_NH = 12


def _erf_approx(x):
    a1, a2, a3, a4, a5 = (0.254829592, -0.284496736, 1.421413741,
                          -1.453152027, 1.061405429)
    p = 0.3275911
    ax = jnp.abs(x)
    t = 1.0 / (1.0 + p * ax)
    poly = ((((a5 * t + a4) * t + a3) * t + a2) * t + a1) * t
    y = 1.0 - poly * jnp.exp(-ax * ax)
    return jnp.where(x >= 0, y, -y)


def _gelu(x):
    return x * 0.5 * (1.0 + _erf_approx(x * (1.0 / math.sqrt(2.0))))


def _ln(x, g, b):
    u = jnp.mean(x, axis=-1, keepdims=True)
    m2 = jnp.mean(x * x, axis=-1, keepdims=True)
    inv = jax.lax.rsqrt(m2 - u * u + _EPS)
    return (x - u) * (g * inv) + b


def _bf(x):
    return x.astype(jnp.bfloat16)


def _layer_core(x, m, wqkv, bqkv, wao, bao, gao, beao, wi, bi, wo, bo, go, beo,
                heads):
    S, H = x.shape
    dh = H // heads
    scale = 1.0 / math.sqrt(dh)
    qkv = jnp.dot(_bf(x), wqkv, preferred_element_type=jnp.float32) + bqkv
    parts = []
    for h in range(heads):
        q = qkv[:, h * dh:(h + 1) * dh]
        k = qkv[:, H + h * dh:H + (h + 1) * dh]
        v = qkv[:, 2 * H + h * dh:2 * H + (h + 1) * dh]
        s = jax.lax.dot_general(q, k, (((1,), (1,)), ((), ())),
                                preferred_element_type=jnp.float32) * scale + m
        p = jnp.exp(s - jnp.max(s, axis=-1, keepdims=True))
        p = p / jnp.sum(p, axis=-1, keepdims=True)
        parts.append(jnp.dot(p, v, preferred_element_type=jnp.float32))
    ctx = jnp.concatenate(parts, axis=-1)
    ao = jnp.dot(_bf(ctx), wao, preferred_element_type=jnp.float32) + bao + x
    attn = _ln(ao, gao, beao)
    inter = _gelu(jnp.dot(_bf(attn), wi, preferred_element_type=jnp.float32) + bi)
    out = jnp.dot(_bf(inter), wo, preferred_element_type=jnp.float32) + bo + attn
    return _ln(out, go, beo)


def _mid_kernel(x_ref, m_ref, *refs, heads):
    *wrefs, o_ref = refs
    o_ref[...] = _layer_core(x_ref[...], m_ref[0],
                             *[r[...] for r in wrefs], heads)


def _first_kernel(wd_ref, ps_ref, tp_ref, ge_ref, bee_ref, m_ref, *refs, heads):
    *wrefs, o_ref = refs
    x = _ln(wd_ref[...] + ps_ref[...] + tp_ref[...], ge_ref[...], bee_ref[...])
    o_ref[...] = _layer_core(x, m_ref[0], *[r[...] for r in wrefs], heads)


def _last_kernel(x_ref, m_ref, *refs, heads):
    *wrefs, o_ref, p_ref = refs
    wpool_ref, bpool_ref = wrefs[-2], wrefs[-1]
    out = _layer_core(x_ref[...], m_ref[0],
                      *[r[...] for r in wrefs[:-2]], heads)
    o_ref[...] = out
    p_ref[0] = jnp.tanh(jnp.dot(_bf(out[0:1, :]), wpool_ref[...],
                                preferred_element_type=jnp.float32)
                        + bpool_ref[...])


def _const2d(shape):
    return pl.BlockSpec(shape, lambda b: (0, 0))


def _layer_specs(H, I):
    return [
        _const2d((H, 3 * H)), _const2d((1, 3 * H)),
        _const2d((H, H)), _const2d((1, H)), _const2d((1, H)), _const2d((1, H)),
        _const2d((H, I)), _const2d((1, I)),
        _const2d((I, H)), _const2d((1, H)), _const2d((1, H)), _const2d((1, H)),
    ]


def _params():
    return pltpu.CompilerParams(
        dimension_semantics=("arbitrary",),
        vmem_limit_bytes=_VMEM_LIMIT,
    )


def _pack_layer(w_qkv, b_qkv, w_ao, b_ao, g_ao, be_ao, w_i, b_i, w_o, b_o,
                g_o, be_o):
    H = w_qkv.shape[0]
    I = w_i.shape[1]
    return (_bf(w_qkv), b_qkv.reshape(1, 3 * H),
            _bf(w_ao), b_ao.reshape(1, H), g_ao.reshape(1, H),
            be_ao.reshape(1, H),
            _bf(w_i), b_i.reshape(1, I),
            _bf(w_o), b_o.reshape(1, H), g_o.reshape(1, H), be_o.reshape(1, H))


def kernel(word_emb, pos_emb, type_emb, emb_gamma, emb_beta, w_pool, b_pool,
           input_ids, token_type_ids, attention_mask,
           L0_w_qkv, L0_b_qkv, L0_w_ao, L0_b_ao, L0_g_ao, L0_be_ao,
           L0_w_i, L0_b_i, L0_w_o, L0_b_o, L0_g_o, L0_be_o,
           L1_w_qkv, L1_b_qkv, L1_w_ao, L1_b_ao, L1_g_ao, L1_be_ao,
           L1_w_i, L1_b_i, L1_w_o, L1_b_o, L1_g_o, L1_be_o,
           L2_w_qkv, L2_b_qkv, L2_w_ao, L2_b_ao, L2_g_ao, L2_be_ao,
           L2_w_i, L2_b_i, L2_w_o, L2_b_o, L2_g_o, L2_be_o,
           L3_w_qkv, L3_b_qkv, L3_w_ao, L3_b_ao, L3_g_ao, L3_be_ao,
           L3_w_i, L3_b_i, L3_w_o, L3_b_o, L3_g_o, L3_be_o,
           L4_w_qkv, L4_b_qkv, L4_w_ao, L4_b_ao, L4_g_ao, L4_be_ao,
           L4_w_i, L4_b_i, L4_w_o, L4_b_o, L4_g_o, L4_be_o,
           L5_w_qkv, L5_b_qkv, L5_w_ao, L5_b_ao, L5_g_ao, L5_be_ao,
           L5_w_i, L5_b_i, L5_w_o, L5_b_o, L5_g_o, L5_be_o):
    _flat = [
        L0_w_qkv, L0_b_qkv, L0_w_ao, L0_b_ao, L0_g_ao, L0_be_ao,
        L0_w_i, L0_b_i, L0_w_o, L0_b_o, L0_g_o, L0_be_o,
        L1_w_qkv, L1_b_qkv, L1_w_ao, L1_b_ao, L1_g_ao, L1_be_ao,
        L1_w_i, L1_b_i, L1_w_o, L1_b_o, L1_g_o, L1_be_o,
        L2_w_qkv, L2_b_qkv, L2_w_ao, L2_b_ao, L2_g_ao, L2_be_ao,
        L2_w_i, L2_b_i, L2_w_o, L2_b_o, L2_g_o, L2_be_o,
        L3_w_qkv, L3_b_qkv, L3_w_ao, L3_b_ao, L3_g_ao, L3_be_ao,
        L3_w_i, L3_b_i, L3_w_o, L3_b_o, L3_g_o, L3_be_o,
        L4_w_qkv, L4_b_qkv, L4_w_ao, L4_b_ao, L4_g_ao, L4_be_ao,
        L4_w_i, L4_b_i, L4_w_o, L4_b_o, L4_g_o, L4_be_o,
        L5_w_qkv, L5_b_qkv, L5_w_ao, L5_b_ao, L5_g_ao, L5_be_ao,
        L5_w_i, L5_b_i, L5_w_o, L5_b_o, L5_g_o, L5_be_o,
    ]
    layers = [_pack_layer(*_flat[i * 12:(i + 1) * 12]) for i in range(6)]
    B, S = input_ids.shape
    H = word_emb.shape[1]
    I = L0_w_i.shape[1]
    M = B * S
    heads = _NH

    ext = (1.0 - attention_mask.astype(jnp.float32)) * -1000000.0
    mask3 = ext.reshape(B, 1, S)
    words = jnp.take(word_emb, input_ids.reshape(-1), axis=0)
    types = jnp.take(type_emb, token_type_ids.reshape(-1), axis=0)
    pos = pos_emb[:S]

    wspecs = _layer_specs(H, I)
    row = pl.BlockSpec((S, H), lambda b: (b, 0))
    mspec = pl.BlockSpec((1, 1, S), lambda b: (b, 0, 0))
    vspec = _const2d((1, H))

    h0 = pl.pallas_call(
        functools.partial(_first_kernel, heads=heads),
        out_shape=jax.ShapeDtypeStruct((M, H), jnp.float32),
        grid=(B,),
        in_specs=[row, _const2d((S, H)), row, vspec, vspec, mspec] + wspecs,
        out_specs=row,
        compiler_params=_params(),
    )(words, pos, types, emb_gamma.reshape(1, H), emb_beta.reshape(1, H),
      mask3, *layers[0])

    hs = [h0]
    for li in range(1, 5):
        hs.append(pl.pallas_call(
            functools.partial(_mid_kernel, heads=heads),
            out_shape=jax.ShapeDtypeStruct((M, H), jnp.float32),
            grid=(B,),
            in_specs=[row, mspec] + wspecs,
            out_specs=row,
            compiler_params=_params(),
        )(hs[-1], mask3, *layers[li]))

    h5, pooled3 = pl.pallas_call(
        functools.partial(_last_kernel, heads=heads),
        out_shape=[jax.ShapeDtypeStruct((M, H), jnp.float32),
                   jax.ShapeDtypeStruct((B, 1, H), jnp.float32)],
        grid=(B,),
        in_specs=[row, mspec] + wspecs + [_const2d((H, H)), vspec],
        out_specs=[row, pl.BlockSpec((1, 1, H), lambda b: (b, 0, 0))],
        compiler_params=_params(),
    )(hs[-1], mask3, *layers[5], _bf(w_pool), b_pool.reshape(1, H))
    hs.append(h5)

    all_layers = [h.reshape(B, S, H) for h in hs]
    return all_layers, pooled3.reshape(B, H)


# speedup vs baseline: 3.0644x; 1.3583x over previous
import functools
import math

import jax
import jax.numpy as jnp
from jax.experimental import pallas as pl
from jax.experimental.pallas import tpu as pltpu

_EPS = 1e-12
_VMEM_LIMIT = 64 * 1024 * 1024
_NH = 12


def _erf_approx(x):
    a1, a2, a3, a4, a5 = (0.254829592, -0.284496736, 1.421413741,
                          -1.453152027, 1.061405429)
    p = 0.3275911
    ax = jnp.abs(x)
    t = 1.0 / (1.0 + p * ax)
    poly = ((((a5 * t + a4) * t + a3) * t + a2) * t + a1) * t
    y = 1.0 - poly * jnp.exp(-ax * ax)
    return jnp.where(x >= 0, y, -y)


def _gelu(x):
    return x * 0.5 * (1.0 + _erf_approx(x * (1.0 / math.sqrt(2.0))))


def _ln(x, g, b):
    u = jnp.mean(x, axis=-1, keepdims=True)
    m2 = jnp.mean(x * x, axis=-1, keepdims=True)
    inv = jax.lax.rsqrt(m2 - u * u + _EPS)
    return (x - u) * (g * inv) + b


def _bf(x):
    return x.astype(jnp.bfloat16)


def _layer_core(x, m, wqkv, bqkv, wao, bao, gao, beao, wi, bi, wo, bo, go, beo,
                heads):
    S, H = x.shape
    dh = H // heads
    scale = 1.0 / math.sqrt(dh)
    qkv = jnp.dot(_bf(x), wqkv, preferred_element_type=jnp.float32) + bqkv

    def _qk(h):
        q = qkv[:, h * dh:(h + 1) * dh]
        k = qkv[:, H + h * dh:H + (h + 1) * dh]
        return jax.lax.dot_general(q, k, (((1,), (1,)), ((), ())),
                                   preferred_element_type=jnp.float32)

    s_all = jnp.concatenate([_qk(h) for h in range(heads)], axis=0)
    s_all = s_all * scale + m
    p = jnp.exp(s_all - jnp.max(s_all, axis=-1, keepdims=True))
    p = p / jnp.sum(p, axis=-1, keepdims=True)
    parts = [jnp.dot(p[h * S:(h + 1) * S, :],
                     qkv[:, 2 * H + h * dh:2 * H + (h + 1) * dh],
                     preferred_element_type=jnp.float32)
             for h in range(heads)]
    ctx = jnp.concatenate(parts, axis=-1)
    ao = jnp.dot(_bf(ctx), wao, preferred_element_type=jnp.float32) + bao + x
    attn = _ln(ao, gao, beao)
    inter = _gelu(jnp.dot(_bf(attn), wi, preferred_element_type=jnp.float32) + bi)
    out = jnp.dot(_bf(inter), wo, preferred_element_type=jnp.float32) + bo + attn
    return _ln(out, go, beo)


def _mid_kernel(x_ref, m_ref, *refs, heads):
    *wrefs, o_ref = refs
    o_ref[...] = _layer_core(x_ref[...], m_ref[0],
                             *[r[...] for r in wrefs], heads)


def _first_kernel(wd_ref, ps_ref, tp_ref, ge_ref, bee_ref, m_ref, *refs, heads):
    *wrefs, o_ref = refs
    x = _ln(wd_ref[...] + ps_ref[...] + tp_ref[...], ge_ref[...], bee_ref[...])
    o_ref[...] = _layer_core(x, m_ref[0], *[r[...] for r in wrefs], heads)


def _last_kernel(x_ref, m_ref, *refs, heads):
    *wrefs, o_ref, p_ref = refs
    wpool_ref, bpool_ref = wrefs[-2], wrefs[-1]
    out = _layer_core(x_ref[...], m_ref[0],
                      *[r[...] for r in wrefs[:-2]], heads)
    o_ref[...] = out
    p_ref[0] = jnp.tanh(jnp.dot(_bf(out[0:1, :]), wpool_ref[...],
                                preferred_element_type=jnp.float32)
                        + bpool_ref[...])


def _const2d(shape):
    return pl.BlockSpec(shape, lambda b: (0, 0))


def _layer_specs(H, I):
    return [
        _const2d((H, 3 * H)), _const2d((1, 3 * H)),
        _const2d((H, H)), _const2d((1, H)), _const2d((1, H)), _const2d((1, H)),
        _const2d((H, I)), _const2d((1, I)),
        _const2d((I, H)), _const2d((1, H)), _const2d((1, H)), _const2d((1, H)),
    ]


def _params():
    return pltpu.CompilerParams(
        dimension_semantics=("arbitrary",),
        vmem_limit_bytes=_VMEM_LIMIT,
    )


def _pack_layer(w_qkv, b_qkv, w_ao, b_ao, g_ao, be_ao, w_i, b_i, w_o, b_o,
                g_o, be_o):
    H = w_qkv.shape[0]
    I = w_i.shape[1]
    return (_bf(w_qkv), b_qkv.reshape(1, 3 * H),
            _bf(w_ao), b_ao.reshape(1, H), g_ao.reshape(1, H),
            be_ao.reshape(1, H),
            _bf(w_i), b_i.reshape(1, I),
            _bf(w_o), b_o.reshape(1, H), g_o.reshape(1, H), be_o.reshape(1, H))


def kernel(word_emb, pos_emb, type_emb, emb_gamma, emb_beta, w_pool, b_pool,
           input_ids, token_type_ids, attention_mask,
           L0_w_qkv, L0_b_qkv, L0_w_ao, L0_b_ao, L0_g_ao, L0_be_ao,
           L0_w_i, L0_b_i, L0_w_o, L0_b_o, L0_g_o, L0_be_o,
           L1_w_qkv, L1_b_qkv, L1_w_ao, L1_b_ao, L1_g_ao, L1_be_ao,
           L1_w_i, L1_b_i, L1_w_o, L1_b_o, L1_g_o, L1_be_o,
           L2_w_qkv, L2_b_qkv, L2_w_ao, L2_b_ao, L2_g_ao, L2_be_ao,
           L2_w_i, L2_b_i, L2_w_o, L2_b_o, L2_g_o, L2_be_o,
           L3_w_qkv, L3_b_qkv, L3_w_ao, L3_b_ao, L3_g_ao, L3_be_ao,
           L3_w_i, L3_b_i, L3_w_o, L3_b_o, L3_g_o, L3_be_o,
           L4_w_qkv, L4_b_qkv, L4_w_ao, L4_b_ao, L4_g_ao, L4_be_ao,
           L4_w_i, L4_b_i, L4_w_o, L4_b_o, L4_g_o, L4_be_o,
           L5_w_qkv, L5_b_qkv, L5_w_ao, L5_b_ao, L5_g_ao, L5_be_ao,
           L5_w_i, L5_b_i, L5_w_o, L5_b_o, L5_g_o, L5_be_o):
    _flat = [
        L0_w_qkv, L0_b_qkv, L0_w_ao, L0_b_ao, L0_g_ao, L0_be_ao,
        L0_w_i, L0_b_i, L0_w_o, L0_b_o, L0_g_o, L0_be_o,
        L1_w_qkv, L1_b_qkv, L1_w_ao, L1_b_ao, L1_g_ao, L1_be_ao,
        L1_w_i, L1_b_i, L1_w_o, L1_b_o, L1_g_o, L1_be_o,
        L2_w_qkv, L2_b_qkv, L2_w_ao, L2_b_ao, L2_g_ao, L2_be_ao,
        L2_w_i, L2_b_i, L2_w_o, L2_b_o, L2_g_o, L2_be_o,
        L3_w_qkv, L3_b_qkv, L3_w_ao, L3_b_ao, L3_g_ao, L3_be_ao,
        L3_w_i, L3_b_i, L3_w_o, L3_b_o, L3_g_o, L3_be_o,
        L4_w_qkv, L4_b_qkv, L4_w_ao, L4_b_ao, L4_g_ao, L4_be_ao,
        L4_w_i, L4_b_i, L4_w_o, L4_b_o, L4_g_o, L4_be_o,
        L5_w_qkv, L5_b_qkv, L5_w_ao, L5_b_ao, L5_g_ao, L5_be_ao,
        L5_w_i, L5_b_i, L5_w_o, L5_b_o, L5_g_o, L5_be_o,
    ]
    layers = [_pack_layer(*_flat[i * 12:(i + 1) * 12]) for i in range(6)]
    B, S = input_ids.shape
    H = word_emb.shape[1]
    I = L0_w_i.shape[1]
    M = B * S
    heads = _NH

    ext = (1.0 - attention_mask.astype(jnp.float32)) * -1000000.0
    mask3 = ext.reshape(B, 1, S)
    words = jnp.take(word_emb, input_ids.reshape(-1), axis=0)
    types = jnp.take(type_emb, token_type_ids.reshape(-1), axis=0)
    pos = pos_emb[:S]

    wspecs = _layer_specs(H, I)
    row = pl.BlockSpec((S, H), lambda b: (b, 0))
    mspec = pl.BlockSpec((1, 1, S), lambda b: (b, 0, 0))
    vspec = _const2d((1, H))

    h0 = pl.pallas_call(
        functools.partial(_first_kernel, heads=heads),
        out_shape=jax.ShapeDtypeStruct((M, H), jnp.float32),
        grid=(B,),
        in_specs=[row, _const2d((S, H)), row, vspec, vspec, mspec] + wspecs,
        out_specs=row,
        compiler_params=_params(),
    )(words, pos, types, emb_gamma.reshape(1, H), emb_beta.reshape(1, H),
      mask3, *layers[0])

    hs = [h0]
    for li in range(1, 5):
        hs.append(pl.pallas_call(
            functools.partial(_mid_kernel, heads=heads),
            out_shape=jax.ShapeDtypeStruct((M, H), jnp.float32),
            grid=(B,),
            in_specs=[row, mspec] + wspecs,
            out_specs=row,
            compiler_params=_params(),
        )(hs[-1], mask3, *layers[li]))

    h5, pooled3 = pl.pallas_call(
        functools.partial(_last_kernel, heads=heads),
        out_shape=[jax.ShapeDtypeStruct((M, H), jnp.float32),
                   jax.ShapeDtypeStruct((B, 1, H), jnp.float32)],
        grid=(B,),
        in_specs=[row, mspec] + wspecs + [_const2d((H, H)), vspec],
        out_specs=[row, pl.BlockSpec((1, 1, H), lambda b: (b, 0, 0))],
        compiler_params=_params(),
    )(hs[-1], mask3, *layers[5], _bf(w_pool), b_pool.reshape(1, H))
    hs.append(h5)

    all_layers = [h.reshape(B, S, H) for h in hs]
    return all_layers, pooled3.reshape(B, H)


# speedup vs baseline: 3.3473x; 1.0923x over previous
import functools
import math

import jax
import jax.numpy as jnp
from jax.experimental import pallas as pl
from jax.experimental.pallas import tpu as pltpu

_EPS = 1e-12
_VMEM_LIMIT = 64 * 1024 * 1024
_NH = 12


def _erf_approx(x):
    a1, a2, a3, a4, a5 = (0.254829592, -0.284496736, 1.421413741,
                          -1.453152027, 1.061405429)
    p = 0.3275911
    ax = jnp.abs(x)
    t = 1.0 / (1.0 + p * ax)
    poly = ((((a5 * t + a4) * t + a3) * t + a2) * t + a1) * t
    y = 1.0 - poly * jnp.exp(-ax * ax)
    return jnp.where(x >= 0, y, -y)


def _gelu(x):
    return x * 0.5 * (1.0 + _erf_approx(x * (1.0 / math.sqrt(2.0))))


def _ln(x, g, b):
    u = jnp.mean(x, axis=-1, keepdims=True)
    m2 = jnp.mean(x * x, axis=-1, keepdims=True)
    inv = jax.lax.rsqrt(m2 - u * u + _EPS)
    return (x - u) * (g * inv) + b


def _bf(x):
    return x.astype(jnp.bfloat16)


def _layer_core(x, m, wqkv, bqkv, wao, bao, gao, beao, wi, bi, wo, bo, go, beo,
                heads, S):
    Msub, H = x.shape
    bp = Msub // S
    dh = H // heads
    scale = 1.0 / math.sqrt(dh)
    qkv = jnp.dot(_bf(x), wqkv, preferred_element_type=jnp.float32) + bqkv

    def _qk(g, h):
        q = qkv[g * S:(g + 1) * S, h * dh:(h + 1) * dh]
        k = qkv[g * S:(g + 1) * S, H + h * dh:H + (h + 1) * dh]
        return jax.lax.dot_general(q, k, (((1,), (1,)), ((), ())),
                                   preferred_element_type=jnp.float32)

    s_all = jnp.concatenate([_qk(g, h)
                             for g in range(bp) for h in range(heads)], axis=0)
    m_rows = jnp.concatenate([jnp.broadcast_to(m[g], (heads * S, S))
                              for g in range(bp)], axis=0)
    s_all = s_all * scale + m_rows
    p = jnp.exp(s_all - jnp.max(s_all, axis=-1, keepdims=True))
    p = p / jnp.sum(p, axis=-1, keepdims=True)
    ctx = jnp.concatenate(
        [jnp.concatenate(
            [jnp.dot(p[(g * heads + h) * S:(g * heads + h + 1) * S, :],
                     qkv[g * S:(g + 1) * S, 2 * H + h * dh:2 * H + (h + 1) * dh],
                     preferred_element_type=jnp.float32)
             for h in range(heads)], axis=-1)
         for g in range(bp)], axis=0)
    ao = jnp.dot(_bf(ctx), wao, preferred_element_type=jnp.float32) + bao + x
    attn = _ln(ao, gao, beao)
    inter = _gelu(jnp.dot(_bf(attn), wi, preferred_element_type=jnp.float32) + bi)
    out = jnp.dot(_bf(inter), wo, preferred_element_type=jnp.float32) + bo + attn
    return _ln(out, go, beo)


def _mid_kernel(x_ref, m_ref, *refs, heads, S):
    *wrefs, o_ref = refs
    o_ref[...] = _layer_core(x_ref[...], m_ref[...],
                             *[r[...] for r in wrefs], heads, S)


def _first_kernel(wd_ref, ps_ref, tp_ref, ge_ref, bee_ref, m_ref, *refs,
                  heads, S):
    *wrefs, o_ref = refs
    x = _ln(wd_ref[...] + ps_ref[...] + tp_ref[...], ge_ref[...], bee_ref[...])
    o_ref[...] = _layer_core(x, m_ref[...], *[r[...] for r in wrefs], heads, S)


def _last_kernel(x_ref, m_ref, *refs, heads, S):
    *wrefs, o_ref, p_ref = refs
    wpool_ref, bpool_ref = wrefs[-2], wrefs[-1]
    out = _layer_core(x_ref[...], m_ref[...],
                      *[r[...] for r in wrefs[:-2]], heads, S)
    o_ref[...] = out
    bp = m_ref.shape[0]
    first = jnp.concatenate([out[g * S:g * S + 1, :] for g in range(bp)],
                            axis=0)
    pooled = jnp.tanh(jnp.dot(_bf(first), wpool_ref[...],
                              preferred_element_type=jnp.float32)
                      + bpool_ref[...])
    p_ref[...] = pooled.reshape(bp, 1, pooled.shape[-1])


def _const2d(shape):
    return pl.BlockSpec(shape, lambda b: (0, 0))


def _layer_specs(H, I):
    return [
        _const2d((H, 3 * H)), _const2d((1, 3 * H)),
        _const2d((H, H)), _const2d((1, H)), _const2d((1, H)), _const2d((1, H)),
        _const2d((H, I)), _const2d((1, I)),
        _const2d((I, H)), _const2d((1, H)), _const2d((1, H)), _const2d((1, H)),
    ]


def _params():
    return pltpu.CompilerParams(
        dimension_semantics=("arbitrary",),
        vmem_limit_bytes=_VMEM_LIMIT,
    )


def _pack_layer(w_qkv, b_qkv, w_ao, b_ao, g_ao, be_ao, w_i, b_i, w_o, b_o,
                g_o, be_o):
    H = w_qkv.shape[0]
    I = w_i.shape[1]
    return (_bf(w_qkv), b_qkv.reshape(1, 3 * H),
            _bf(w_ao), b_ao.reshape(1, H), g_ao.reshape(1, H),
            be_ao.reshape(1, H),
            _bf(w_i), b_i.reshape(1, I),
            _bf(w_o), b_o.reshape(1, H), g_o.reshape(1, H), be_o.reshape(1, H))


def kernel(word_emb, pos_emb, type_emb, emb_gamma, emb_beta, w_pool, b_pool,
           input_ids, token_type_ids, attention_mask,
           L0_w_qkv, L0_b_qkv, L0_w_ao, L0_b_ao, L0_g_ao, L0_be_ao,
           L0_w_i, L0_b_i, L0_w_o, L0_b_o, L0_g_o, L0_be_o,
           L1_w_qkv, L1_b_qkv, L1_w_ao, L1_b_ao, L1_g_ao, L1_be_ao,
           L1_w_i, L1_b_i, L1_w_o, L1_b_o, L1_g_o, L1_be_o,
           L2_w_qkv, L2_b_qkv, L2_w_ao, L2_b_ao, L2_g_ao, L2_be_ao,
           L2_w_i, L2_b_i, L2_w_o, L2_b_o, L2_g_o, L2_be_o,
           L3_w_qkv, L3_b_qkv, L3_w_ao, L3_b_ao, L3_g_ao, L3_be_ao,
           L3_w_i, L3_b_i, L3_w_o, L3_b_o, L3_g_o, L3_be_o,
           L4_w_qkv, L4_b_qkv, L4_w_ao, L4_b_ao, L4_g_ao, L4_be_ao,
           L4_w_i, L4_b_i, L4_w_o, L4_b_o, L4_g_o, L4_be_o,
           L5_w_qkv, L5_b_qkv, L5_w_ao, L5_b_ao, L5_g_ao, L5_be_ao,
           L5_w_i, L5_b_i, L5_w_o, L5_b_o, L5_g_o, L5_be_o):
    _flat = [
        L0_w_qkv, L0_b_qkv, L0_w_ao, L0_b_ao, L0_g_ao, L0_be_ao,
        L0_w_i, L0_b_i, L0_w_o, L0_b_o, L0_g_o, L0_be_o,
        L1_w_qkv, L1_b_qkv, L1_w_ao, L1_b_ao, L1_g_ao, L1_be_ao,
        L1_w_i, L1_b_i, L1_w_o, L1_b_o, L1_g_o, L1_be_o,
        L2_w_qkv, L2_b_qkv, L2_w_ao, L2_b_ao, L2_g_ao, L2_be_ao,
        L2_w_i, L2_b_i, L2_w_o, L2_b_o, L2_g_o, L2_be_o,
        L3_w_qkv, L3_b_qkv, L3_w_ao, L3_b_ao, L3_g_ao, L3_be_ao,
        L3_w_i, L3_b_i, L3_w_o, L3_b_o, L3_g_o, L3_be_o,
        L4_w_qkv, L4_b_qkv, L4_w_ao, L4_b_ao, L4_g_ao, L4_be_ao,
        L4_w_i, L4_b_i, L4_w_o, L4_b_o, L4_g_o, L4_be_o,
        L5_w_qkv, L5_b_qkv, L5_w_ao, L5_b_ao, L5_g_ao, L5_be_ao,
        L5_w_i, L5_b_i, L5_w_o, L5_b_o, L5_g_o, L5_be_o,
    ]
    layers = [_pack_layer(*_flat[i * 12:(i + 1) * 12]) for i in range(6)]
    B, S = input_ids.shape
    H = word_emb.shape[1]
    I = L0_w_i.shape[1]
    M = B * S
    heads = _NH

    bp = 2 if B % 2 == 0 else 1
    grid = (B // bp,)
    ext = (1.0 - attention_mask.astype(jnp.float32)) * -1000000.0
    mask3 = ext.reshape(B, 1, S)
    words = jnp.take(word_emb, input_ids.reshape(-1), axis=0)
    types = jnp.take(type_emb, token_type_ids.reshape(-1), axis=0)
    pos = jnp.concatenate([pos_emb[:S]] * bp, axis=0)

    wspecs = _layer_specs(H, I)
    row = pl.BlockSpec((bp * S, H), lambda b: (b, 0))
    mspec = pl.BlockSpec((bp, 1, S), lambda b: (b, 0, 0))
    vspec = _const2d((1, H))

    h0 = pl.pallas_call(
        functools.partial(_first_kernel, heads=heads, S=S),
        out_shape=jax.ShapeDtypeStruct((M, H), jnp.float32),
        grid=grid,
        in_specs=[row, _const2d((bp * S, H)), row, vspec, vspec, mspec]
                 + wspecs,
        out_specs=row,
        compiler_params=_params(),
    )(words, pos, types, emb_gamma.reshape(1, H), emb_beta.reshape(1, H),
      mask3, *layers[0])

    hs = [h0]
    for li in range(1, 5):
        hs.append(pl.pallas_call(
            functools.partial(_mid_kernel, heads=heads, S=S),
            out_shape=jax.ShapeDtypeStruct((M, H), jnp.float32),
            grid=grid,
            in_specs=[row, mspec] + wspecs,
            out_specs=row,
            compiler_params=_params(),
        )(hs[-1], mask3, *layers[li]))

    h5, pooled3 = pl.pallas_call(
        functools.partial(_last_kernel, heads=heads, S=S),
        out_shape=[jax.ShapeDtypeStruct((M, H), jnp.float32),
                   jax.ShapeDtypeStruct((B, 1, H), jnp.float32)],
        grid=grid,
        in_specs=[row, mspec] + wspecs + [_const2d((H, H)), vspec],
        out_specs=[row, pl.BlockSpec((bp, 1, H), lambda b: (b, 0, 0))],
        compiler_params=_params(),
    )(hs[-1], mask3, *layers[5], _bf(w_pool), b_pool.reshape(1, H))
    hs.append(h5)

    all_layers = [h.reshape(B, S, H) for h in hs]
    return all_layers, pooled3.reshape(B, H)
